# baseline (speedup 1.0000x reference)
import functools

import jax
import jax.numpy as jnp
from jax import lax
from jax.experimental import pallas as pl
from jax.experimental.pallas import tpu as pltpu
from jax.experimental.pallas import tpu_sc as plsc

B = 16384
D = 64
VOCAB = 1000000
N_COND = 8
NC = 2
NS = 16
NW = NC * NS
W = 256
WSH = 8
NSP_FULL = VOCAB // W
NSP_ALL = NSP_FULL + 1
LASTW = NSP_FULL * W
SPW = 123
NBUF = 4
WLCAP = 2048
NRING = 16
NG = 16
GSH = 3
GCAP = 256

_sc_mesh = plsc.VectorSubcoreMesh(core_axis_name="c", subcore_axis_name="s")


@functools.partial(
    pl.kernel,
    out_type=jax.ShapeDtypeStruct((B, D), jnp.float32),
    mesh=_sc_mesh,
    scratch_types=[
        pltpu.VMEM((B,), jnp.int32),
        pltpu.VMEM((WLCAP,), jnp.int32),
        pltpu.VMEM((WLCAP,), jnp.int32),
        pltpu.VMEM((NG * GCAP,), jnp.int32),
        pltpu.VMEM((NG * GCAP,), jnp.int32),
        pltpu.VMEM((16,), jnp.int32),
        pltpu.VMEM((64,), jnp.int32),
        pltpu.VMEM((64,), jnp.int32),
        pltpu.VMEM((NBUF, D, W), jnp.float32),
        pltpu.VMEM((D, 128), jnp.float32),
        pltpu.VMEM((NRING, D), jnp.float32),
        pltpu.SemaphoreType.DMA,
        pltpu.SemaphoreType.DMA,
        pltpu.SemaphoreType.DMA,
        pltpu.SemaphoreType.DMA,
        pltpu.SemaphoreType.DMA,
        pltpu.SemaphoreType.DMA,
    ],
    compiler_params=pltpu.CompilerParams(
        use_tc_tiling_on_sc=True, needs_layout_passes=False),
)
def _sc_gather(tt_hbm, tail_hbm, x_hbm, out_hbm, xall_v, wlr_v, wlp_v,
               glr_v, glp_v, gcnt_v, phr_v, php_v,
               pan_v, lastp_v, ring_v, sem0, sem1, sem2, sem3, rsem, lsem):
    wid = lax.axis_index("s") * NC + lax.axis_index("c")
    slo = wid * SPW
    shi_mask = jnp.minimum(slo + SPW, NSP_ALL)
    shi_loop = jnp.minimum(slo + SPW, NSP_FULL)
    sems = (sem0, sem1, sem2, sem3)
    lane = lax.iota(jnp.int32, 16)

    def _issue(sp, b):
        start = pl.multiple_of(sp * W, 128)
        pltpu.async_copy(tt_hbm.at[:, pl.ds(start, W)], pan_v.at[b], sems[b])

    def _drain_panel(b):
        pltpu.make_async_copy(tt_hbm.at[:, pl.ds(0, W)], pan_v.at[b],
                              sems[b]).wait()

    for b in range(NBUF):
        @pl.when(slo + b < shi_loop)
        def _(b=b):
            _issue(slo + b, b)

    pltpu.sync_copy(x_hbm, xall_v)

    def _scan(g, off):
        vec = xall_v[pl.ds(g * 16, 16)]
        sp = lax.shift_right_logical(vec, WSH)
        m = (sp >= slo) & (sp < shi_mask)
        plsc.store_compressed(wlr_v.at[pl.ds(off, 16)], vec, mask=m)
        plsc.store_compressed(wlp_v.at[pl.ds(off, 16)], lane + g * 16, mask=m)
        return off + plsc.all_reduce_population_count(m)[0]

    off = lax.fori_loop(0, B // 16, _scan, 0)
    nv = (off + 15) // 16

    gacc = jnp.zeros((16,), jnp.int32)
    for g in range(NG):
        def _gsel(v, cnt, g=g):
            base = v * 16
            rvec = wlr_v[pl.ds(base, 16)]
            pvec = wlp_v[pl.ds(base, 16)]
            grp = lax.shift_right_logical(
                lax.shift_right_logical(rvec, WSH) - slo, GSH)
            m = (grp == g) & (lane + base < off)
            plsc.store_compressed(glr_v.at[pl.ds(g * GCAP + cnt, 16)],
                                  rvec, mask=m)
            plsc.store_compressed(glp_v.at[pl.ds(g * GCAP + cnt, 16)],
                                  pvec, mask=m)
            return cnt + plsc.all_reduce_population_count(m)[0]

        cntg = lax.fori_loop(0, nv, _gsel, 0)
        gacc = jnp.where(lane == g, cntg, gacc)
    gcnt_v[pl.ds(0, 16)] = gacc

    def _extract(src_ref, col, pos, total):
        slot = lax.rem(total, NRING)
        colv = jnp.full((16,), col, jnp.int32)
        for j in range(D // 16):
            v = plsc.load_gather(src_ref, [lane + 16 * j, colv])
            ring_v[slot, pl.ds(16 * j, 16)] = v

        @pl.when(total >= NRING)
        def _():
            pltpu.make_async_copy(ring_v.at[pl.ds(0, 1)],
                                  out_hbm.at[pl.ds(0, 1)], rsem).wait()

        pltpu.async_copy(ring_v.at[pl.ds(slot, 1)],
                         out_hbm.at[pl.ds(pos, 1)], rsem)
        return total + 1

    def _sp_hits(sp, src_ref, total):
        gidx = lax.shift_right_logical(sp - slo, GSH)
        cntg = plsc.load_gather(gcnt_v, [jnp.full((16,), gidx, jnp.int32)])[0]
        gbase = gidx * GCAP

        def _sel(v, cnt):
            base = v * 16
            rvec = glr_v[pl.ds(gbase + base, 16)]
            pvec = glp_v[pl.ds(gbase + base, 16)]
            m = (lax.shift_right_logical(rvec, WSH) == sp) & (lane + base < cntg)
            plsc.store_compressed(phr_v.at[pl.ds(cnt, 16)], rvec, mask=m)
            plsc.store_compressed(php_v.at[pl.ds(cnt, 16)], pvec, mask=m)
            return cnt + plsc.all_reduce_population_count(m)[0]

        cnt = lax.fori_loop(0, (cntg + 15) // 16, _sel, 0)

        def _grp(mg, tot):
            rvec = phr_v[pl.ds(mg * 16, 16)]
            pvec = php_v[pl.ds(mg * 16, 16)]
            for j in range(16):
                k = mg * 16 + j

                def _one(tot=tot, rvec=rvec, pvec=pvec, j=j):
                    r = rvec[j]
                    return _extract(src_ref, r & (W - 1), pvec[j], tot)

                tot = lax.cond(k < cnt, _one, lambda tot=tot: tot)
            return tot

        return lax.fori_loop(0, (cnt + 15) // 16, _grp, total)

    def _outer(g0, total):
        for b in range(NBUF):
            g = g0 * NBUF + b
            sp = slo + g

            @pl.when(sp < shi_loop)
            def _(b=b, sp=sp):
                _drain_panel(b)

            def _consume(b=b, sp=sp, total=total):
                t = _sp_hits(sp, pan_v.at[b], total)

                @pl.when(sp + NBUF < shi_loop)
                def _():
                    _issue(sp + NBUF, b)

                return t

            total = lax.cond(sp < shi_loop, _consume,
                             lambda total=total: total)
        return total

    n_outer = (SPW + NBUF - 1) // NBUF
    total = lax.fori_loop(0, n_outer, _outer, 0)

    def _last(total):
        pltpu.async_copy(tail_hbm, lastp_v, lsem)
        pltpu.make_async_copy(tail_hbm, lastp_v, lsem).wait()

        def _sel(v, cnt):
            base = v * 16
            rvec = wlr_v[pl.ds(base, 16)]
            pvec = wlp_v[pl.ds(base, 16)]
            m = (rvec >= LASTW) & (lane + base < off)
            plsc.store_compressed(phr_v.at[pl.ds(cnt, 16)], rvec, mask=m)
            plsc.store_compressed(php_v.at[pl.ds(cnt, 16)], pvec, mask=m)
            return cnt + plsc.all_reduce_population_count(m)[0]

        cnt = lax.fori_loop(0, nv, _sel, 0)

        def _grp(mg, tot):
            rvec = phr_v[pl.ds(mg * 16, 16)]
            pvec = php_v[pl.ds(mg * 16, 16)]
            for j in range(16):
                k = mg * 16 + j

                def _one(tot=tot, rvec=rvec, pvec=pvec, j=j):
                    r = rvec[j]
                    return _extract(lastp_v, r - LASTW, pvec[j], tot)

                tot = lax.cond(k < cnt, _one, lambda tot=tot: tot)
            return tot

        return lax.fori_loop(0, (cnt + 15) // 16, _grp, total)

    total = lax.cond(shi_mask == NSP_ALL, _last, lambda t: t, total)

    def _drain_rows(_, c):
        pltpu.make_async_copy(ring_v.at[pl.ds(0, 1)],
                              out_hbm.at[pl.ds(0, 1)], rsem).wait()
        return c

    lax.fori_loop(0, jnp.minimum(total, NRING), _drain_rows, 0)


TC_GRID = 16
TC_BLK = B // TC_GRID


def _tc_body(emb_ref, c_ref, masks_ref, out_ref, mn_ref, en_ref,
             mn_acc, en_acc):
    i = pl.program_id(0)
    emb = emb_ref[...]
    cc = c_ref[...]
    m = jnp.maximum(masks_ref[...], 0.0)
    iota = lax.broadcasted_iota(jnp.int32, (1, N_COND), 1)
    onehot = (cc == iota).astype(jnp.float32)
    mask = lax.dot_general(
        onehot, m, (((1,), (0,)), ((), ())),
        preferred_element_type=jnp.float32,
    )
    masked = emb * mask
    rn = jnp.sqrt(jnp.sum(masked * masked, axis=1, keepdims=True))
    out_ref[...] = masked / jnp.maximum(rn, 1e-10)

    mn_part = jnp.sum(jnp.abs(mask))
    en_part = jnp.sum(emb * emb)

    @pl.when(i == 0)
    def _init():
        mn_acc[0] = mn_part
        en_acc[0] = en_part

    @pl.when(i > 0)
    def _accum():
        mn_acc[0] += mn_part
        en_acc[0] += en_part

    @pl.when(i == TC_GRID - 1)
    def _fin():
        mn_ref[...] = jnp.reshape(mn_acc[0], (1, 1))
        en_ref[...] = jnp.reshape(jnp.sqrt(en_acc[0]), (1, 1))


_tc_call = pl.pallas_call(
    _tc_body,
    grid=(TC_GRID,),
    in_specs=[
        pl.BlockSpec((TC_BLK, D), lambda i: (i, 0)),
        pl.BlockSpec((TC_BLK, 1), lambda i: (i, 0)),
        pl.BlockSpec((N_COND, D), lambda i: (0, 0)),
    ],
    out_specs=[
        pl.BlockSpec((TC_BLK, D), lambda i: (i, 0)),
        pl.BlockSpec((1, 1), lambda i: (0, 0)),
        pl.BlockSpec((1, 1), lambda i: (0, 0)),
    ],
    out_shape=[
        jax.ShapeDtypeStruct((B, D), jnp.float32),
        jax.ShapeDtypeStruct((1, 1), jnp.float32),
        jax.ShapeDtypeStruct((1, 1), jnp.float32),
    ],
    scratch_shapes=[
        pltpu.SMEM((1,), jnp.float32),
        pltpu.SMEM((1,), jnp.float32),
    ],
)


def kernel(x, c, table, masks):
    tt = table.T
    tail = jnp.pad(lax.slice(tt, (0, LASTW), (D, VOCAB)), ((0, 0), (0, 64)))
    embedded = _sc_gather(tt, tail, x.astype(jnp.int32))
    out, mn, en = _tc_call(embedded, c.astype(jnp.int32).reshape(B, 1), masks)
    return out, mn.reshape(()), en.reshape(()), embedded


# speedup vs baseline: 1.2484x; 1.2484x over previous
import functools

import jax
import jax.numpy as jnp
from jax import lax
from jax.experimental import pallas as pl
from jax.experimental.pallas import tpu as pltpu
from jax.experimental.pallas import tpu_sc as plsc

B = 16384
D = 64
VOCAB = 1000000
N_COND = 8
NC = 2
NS = 16
NW = NC * NS
W = 512
NSP_FULL = VOCAB // W
NSP_ALL = NSP_FULL + 1
LASTW = NSP_FULL * W
SPW = 62
NBUF = 2
WLCAP = 2048
NRING = 16
NG = 16
GSH = 2
GCAP = 256

_sc_mesh = plsc.VectorSubcoreMesh(core_axis_name="c", subcore_axis_name="s")


@functools.partial(
    pl.kernel,
    out_type=jax.ShapeDtypeStruct((B, D), jnp.float32),
    mesh=_sc_mesh,
    scratch_types=[
        pltpu.VMEM((B,), jnp.int32),
        pltpu.VMEM((WLCAP,), jnp.int32),
        pltpu.VMEM((WLCAP,), jnp.int32),
        pltpu.VMEM((NG * GCAP,), jnp.int32),
        pltpu.VMEM((NG * GCAP,), jnp.int32),
        pltpu.VMEM((16,), jnp.int32),
        pltpu.VMEM((64,), jnp.int32),
        pltpu.VMEM((64,), jnp.int32),
        pltpu.VMEM((NBUF, D, W), jnp.float32),
        pltpu.VMEM((D, 128), jnp.float32),
        pltpu.VMEM((NRING, D), jnp.float32),
        pltpu.SemaphoreType.DMA,
        pltpu.SemaphoreType.DMA,
        pltpu.SemaphoreType.DMA,
        pltpu.SemaphoreType.DMA,
    ],
    compiler_params=pltpu.CompilerParams(
        use_tc_tiling_on_sc=True, needs_layout_passes=False),
)
def _sc_gather(tt_hbm, tail_hbm, x_hbm, out_hbm, xall_v, wlr_v, wlp_v,
               glr_v, glp_v, gcnt_v, phr_v, php_v,
               pan_v, lastp_v, ring_v, sem0, sem1, rsem, lsem):
    wid = lax.axis_index("s") * NC + lax.axis_index("c")
    slo = wid * SPW
    shi_mask = jnp.minimum(slo + SPW, NSP_ALL)
    shi_loop = jnp.minimum(slo + SPW, NSP_FULL)
    sems = (sem0, sem1)
    lane = lax.iota(jnp.int32, 16)

    def _issue(sp, b):
        start = pl.multiple_of(sp * W, 128)
        pltpu.async_copy(tt_hbm.at[:, pl.ds(start, W)], pan_v.at[b], sems[b])

    def _drain_panel(b):
        pltpu.make_async_copy(tt_hbm.at[:, pl.ds(0, W)], pan_v.at[b],
                              sems[b]).wait()

    for b in range(NBUF):
        @pl.when(slo + b < shi_loop)
        def _(b=b):
            _issue(slo + b, b)

    pltpu.sync_copy(x_hbm, xall_v)

    def _scan(g, off):
        vec = xall_v[pl.ds(g * 16, 16)]
        sp = lax.shift_right_logical(vec, 9)
        m = (sp >= slo) & (sp < shi_mask)
        plsc.store_compressed(wlr_v.at[pl.ds(off, 16)], vec, mask=m)
        plsc.store_compressed(wlp_v.at[pl.ds(off, 16)], lane + g * 16, mask=m)
        return off + plsc.all_reduce_population_count(m)[0]

    off = lax.fori_loop(0, B // 16, _scan, 0)
    nv = (off + 15) // 16

    gacc = jnp.zeros((16,), jnp.int32)
    for g in range(NG):
        def _gsel(v, cnt, g=g):
            base = v * 16
            rvec = wlr_v[pl.ds(base, 16)]
            pvec = wlp_v[pl.ds(base, 16)]
            grp = lax.shift_right_logical(
                lax.shift_right_logical(rvec, 9) - slo, GSH)
            m = (grp == g) & (lane + base < off)
            plsc.store_compressed(glr_v.at[pl.ds(g * GCAP + cnt, 16)],
                                  rvec, mask=m)
            plsc.store_compressed(glp_v.at[pl.ds(g * GCAP + cnt, 16)],
                                  pvec, mask=m)
            return cnt + plsc.all_reduce_population_count(m)[0]

        cntg = lax.fori_loop(0, nv, _gsel, 0)
        gacc = jnp.where(lane == g, cntg, gacc)
    gcnt_v[pl.ds(0, 16)] = gacc

    def _extract(src_ref, col, pos, total):
        slot = lax.rem(total, NRING)
        colv = jnp.full((16,), col, jnp.int32)
        for j in range(D // 16):
            v = plsc.load_gather(src_ref, [lane + 16 * j, colv])
            ring_v[slot, pl.ds(16 * j, 16)] = v

        @pl.when(total >= NRING)
        def _():
            pltpu.make_async_copy(ring_v.at[pl.ds(0, 1)],
                                  out_hbm.at[pl.ds(0, 1)], rsem).wait()

        pltpu.async_copy(ring_v.at[pl.ds(slot, 1)],
                         out_hbm.at[pl.ds(pos, 1)], rsem)
        return total + 1

    def _sp_hits(sp, src_ref, total):
        gidx = lax.shift_right_logical(sp - slo, GSH)
        cntg = plsc.load_gather(gcnt_v, [jnp.full((16,), gidx, jnp.int32)])[0]
        gbase = gidx * GCAP

        def _sel(v, cnt):
            base = v * 16
            rvec = glr_v[pl.ds(gbase + base, 16)]
            pvec = glp_v[pl.ds(gbase + base, 16)]
            m = (lax.shift_right_logical(rvec, 9) == sp) & (lane + base < cntg)
            plsc.store_compressed(phr_v.at[pl.ds(cnt, 16)], rvec, mask=m)
            plsc.store_compressed(php_v.at[pl.ds(cnt, 16)], pvec, mask=m)
            return cnt + plsc.all_reduce_population_count(m)[0]

        cnt = lax.fori_loop(0, (cntg + 15) // 16, _sel, 0)

        def _grp(mg, tot):
            rvec = phr_v[pl.ds(mg * 16, 16)]
            pvec = php_v[pl.ds(mg * 16, 16)]
            for j in range(16):
                k = mg * 16 + j

                def _one(tot=tot, rvec=rvec, pvec=pvec, j=j):
                    r = rvec[j]
                    return _extract(src_ref, r & (W - 1), pvec[j], tot)

                tot = lax.cond(k < cnt, _one, lambda tot=tot: tot)
            return tot

        return lax.fori_loop(0, (cnt + 15) // 16, _grp, total)

    def _outer(g0, total):
        for b in range(NBUF):
            g = g0 * NBUF + b
            sp = slo + g

            @pl.when(sp < shi_loop)
            def _(b=b, sp=sp):
                _drain_panel(b)

            def _consume(b=b, sp=sp, total=total):
                t = _sp_hits(sp, pan_v.at[b], total)

                @pl.when(sp + NBUF < shi_loop)
                def _():
                    _issue(sp + NBUF, b)

                return t

            total = lax.cond(sp < shi_loop, _consume,
                             lambda total=total: total)
        return total

    n_outer = (SPW + NBUF - 1) // NBUF
    total = lax.fori_loop(0, n_outer, _outer, 0)

    def _last(total):
        pltpu.async_copy(tail_hbm, lastp_v, lsem)
        pltpu.make_async_copy(tail_hbm, lastp_v, lsem).wait()

        def _sel(v, cnt):
            base = v * 16
            rvec = wlr_v[pl.ds(base, 16)]
            pvec = wlp_v[pl.ds(base, 16)]
            m = (rvec >= LASTW) & (lane + base < off)
            plsc.store_compressed(phr_v.at[pl.ds(cnt, 16)], rvec, mask=m)
            plsc.store_compressed(php_v.at[pl.ds(cnt, 16)], pvec, mask=m)
            return cnt + plsc.all_reduce_population_count(m)[0]

        cnt = lax.fori_loop(0, nv, _sel, 0)

        def _grp(mg, tot):
            rvec = phr_v[pl.ds(mg * 16, 16)]
            pvec = php_v[pl.ds(mg * 16, 16)]
            for j in range(16):
                k = mg * 16 + j

                def _one(tot=tot, rvec=rvec, pvec=pvec, j=j):
                    r = rvec[j]
                    return _extract(lastp_v, r - LASTW, pvec[j], tot)

                tot = lax.cond(k < cnt, _one, lambda tot=tot: tot)
            return tot

        return lax.fori_loop(0, (cnt + 15) // 16, _grp, total)

    total = lax.cond(shi_mask == NSP_ALL, _last, lambda t: t, total)

    def _drain_rows(_, c):
        pltpu.make_async_copy(ring_v.at[pl.ds(0, 1)],
                              out_hbm.at[pl.ds(0, 1)], rsem).wait()
        return c

    lax.fori_loop(0, jnp.minimum(total, NRING), _drain_rows, 0)


TC_GRID = 16
TC_BLK = B // TC_GRID


def _tc_body(emb_ref, c_ref, masks_ref, out_ref, mn_ref, en_ref,
             mn_acc, en_acc):
    i = pl.program_id(0)
    emb = emb_ref[...]
    cc = c_ref[...]
    m = jnp.maximum(masks_ref[...], 0.0)
    iota = lax.broadcasted_iota(jnp.int32, (1, N_COND), 1)
    onehot = (cc == iota).astype(jnp.float32)
    mask = lax.dot_general(
        onehot, m, (((1,), (0,)), ((), ())),
        preferred_element_type=jnp.float32,
    )
    masked = emb * mask
    rn = jnp.sqrt(jnp.sum(masked * masked, axis=1, keepdims=True))
    out_ref[...] = masked / jnp.maximum(rn, 1e-10)

    mn_part = jnp.sum(jnp.abs(mask))
    en_part = jnp.sum(emb * emb)

    @pl.when(i == 0)
    def _init():
        mn_acc[0] = mn_part
        en_acc[0] = en_part

    @pl.when(i > 0)
    def _accum():
        mn_acc[0] += mn_part
        en_acc[0] += en_part

    @pl.when(i == TC_GRID - 1)
    def _fin():
        mn_ref[...] = jnp.reshape(mn_acc[0], (1, 1))
        en_ref[...] = jnp.reshape(jnp.sqrt(en_acc[0]), (1, 1))


_tc_call = pl.pallas_call(
    _tc_body,
    grid=(TC_GRID,),
    in_specs=[
        pl.BlockSpec((TC_BLK, D), lambda i: (i, 0)),
        pl.BlockSpec((TC_BLK, 1), lambda i: (i, 0)),
        pl.BlockSpec((N_COND, D), lambda i: (0, 0)),
    ],
    out_specs=[
        pl.BlockSpec((TC_BLK, D), lambda i: (i, 0)),
        pl.BlockSpec((1, 1), lambda i: (0, 0)),
        pl.BlockSpec((1, 1), lambda i: (0, 0)),
    ],
    out_shape=[
        jax.ShapeDtypeStruct((B, D), jnp.float32),
        jax.ShapeDtypeStruct((1, 1), jnp.float32),
        jax.ShapeDtypeStruct((1, 1), jnp.float32),
    ],
    scratch_shapes=[
        pltpu.SMEM((1,), jnp.float32),
        pltpu.SMEM((1,), jnp.float32),
    ],
)


def kernel(x, c, table, masks):
    tt = table.T
    tail = jnp.pad(lax.slice(tt, (0, LASTW), (D, VOCAB)), ((0, 0), (0, 64)))
    embedded = _sc_gather(tt, tail, x.astype(jnp.int32))
    out, mn, en = _tc_call(embedded, c.astype(jnp.int32).reshape(B, 1), masks)
    return out, mn.reshape(()), en.reshape(()), embedded
